# baseline (speedup 1.0000x reference)
import jax
import jax.numpy as jnp
from jax import lax
from jax.experimental import pallas as pl
from jax.experimental.pallas import tpu as pltpu
from jax.experimental.pallas import tpu_sc as plsc

NUM_USERS = 25000
NUM_NODES = 50000
EMB = 64
DQ = EMB // 4
E = 800000
B = 4096
LAYERS = 3

NC = 2
NS = 16
L = 16

CHUNK = 1024
SUB = 128
NSUB = CHUNK // SUB
EPT = 53248
NCHUNKS = EPT // CHUNK
NPAIRS = NCHUNKS // 2
E_PAD = EPT * NS
EROWS = (NSUB * SUB * 4) // (DQ * 4)
ROWS_PT = 3128
ROWS_LAST = NUM_NODES - (NS - 1) * ROWS_PT

ZB = 1024
_SPANS_COMMON = ((0, ZB), (ZB, ZB), (2 * ZB, ZB),
                 (3 * ZB, ROWS_LAST - 3 * ZB))
_SPAN_EXTRA = (ROWS_LAST, ROWS_PT - ROWS_LAST)

_mesh = plsc.VectorSubcoreMesh(
    core_axis_name="c", subcore_axis_name="s", num_cores=NC, num_subcores=NS
)
_params = pltpu.CompilerParams(
    needs_layout_passes=False, use_tc_tiling_on_sc=False
)


def _layer_body(x0, x1, x2, x3, src_h, dst_h, nrm_h, y0, y1, y2, y3,
                sb0, db0, nb0, rw0, sb1, db1, nb1, rw1,
                sb2, db2, nb2, rw2, sb3, db3, nb3, rw3,
                acc, esem, gsem, ssem):
    allbufs = ((sb0, db0, nb0, rw0), (sb1, db1, nb1, rw1),
               (sb2, db2, nb2, rw2), (sb3, db3, nb3, rw3))
    rows = rw0
    cid = lax.axis_index("c")
    sid = lax.axis_index("s")

    zero16 = jnp.zeros((L,), jnp.float32)
    base_row = sid * ROWS_PT
    erow_base = sid * (EPT // SUB)
    eflat_base = sid * EPT

    xs = (x0, x1, x2, x3)
    ys = (y0, y1, y2, y3)

    for q in range(4):
        @pl.when(cid == q // 2)
        def _(xq=xs[q], yq=ys[q]):
            def _zrow(j, c):
                rows[j, 0:16] = zero16
                return c
            lax.fori_loop(0, ZB, _zrow, 0, unroll=8)
            for off, n in _SPANS_COMMON:
                pltpu.sync_copy(rows.at[pl.ds(0, n)],
                                acc.at[pl.ds(base_row + off, n)])

            @pl.when(sid < NS - 1)
            def _():
                off, n = _SPAN_EXTRA
                pltpu.sync_copy(rows.at[pl.ds(0, n)],
                                acc.at[pl.ds(base_row + off, n)])
            plsc.subcore_barrier()

            def _fire_edges(g, bi):
                sb, db, nb, _ = allbufs[bi]
                rb = erow_base + g * NSUB
                pltpu.async_copy(src_h.at[pl.ds(rb, NSUB)], sb, esem)
                pltpu.async_copy(
                    nrm_h.at[pl.ds(eflat_base + g * CHUNK, CHUNK)], nb, esem)
                pltpu.async_copy(dst_h.at[pl.ds(rb, NSUB)], db, esem)

            def _fire_gathers(bi):
                sb, _, _, rw = allbufs[bi]
                for r in range(NSUB):
                    pltpu.async_copy(xq.at[sb.at[r]],
                                     rw.at[pl.ds(r * SUB, SUB)], gsem)

            def _scale(bi):
                _, _, nb, rw = allbufs[bi]

                def _blk(b, cc):
                    j0 = b * L
                    nvec = nb[pl.ds(j0, L)]
                    for e in range(L):
                        nv = lax.broadcast(nvec[e], (L,))
                        rw[j0 + e, 0:16] = rw[j0 + e, 0:16] * nv
                    return cc
                lax.fori_loop(0, CHUNK // L, _blk, 0, unroll=2)

            def _fire_scatters(bi):
                _, db, _, rw = allbufs[bi]
                for r in range(NSUB):
                    pltpu.async_copy(rw.at[pl.ds(r * SUB, SUB)],
                                     acc.at[db.at[r]], ssem, add=True)

            def _drain_edges():
                pltpu.make_async_copy(xq.at[pl.ds(0, 3 * EROWS)],
                                      rw0.at[pl.ds(0, 3 * EROWS)],
                                      esem).wait()

            def _drain_chunk(sem, rw):
                pltpu.make_async_copy(xq.at[pl.ds(0, CHUNK)], rw, sem).wait()

            def _run_pair(t, s0, s1, u):
                @pl.when(u >= 1)
                def _():
                    _drain_chunk(ssem, allbufs[s0][3])
                    _drain_chunk(ssem, allbufs[s1][3])
                _fire_edges(2 * t, s0)
                _fire_edges(2 * t + 1, s1)
                _drain_edges()
                _fire_gathers(s0)
                _drain_edges()
                _fire_gathers(s1)
                _drain_chunk(gsem, allbufs[s0][3])
                _scale(s0)
                _fire_scatters(s0)
                _drain_chunk(gsem, allbufs[s1][3])
                _scale(s1)
                _fire_scatters(s1)

            def _iter(u, c):
                _run_pair(2 * u, 0, 1, u)
                _run_pair(2 * u + 1, 2, 3, u)
                return c

            lax.fori_loop(0, NPAIRS // 2, _iter, 0)
            for bi in range(4):
                _drain_chunk(ssem, allbufs[bi][3])
            plsc.subcore_barrier()

            for off, n in _SPANS_COMMON:
                pltpu.sync_copy(acc.at[pl.ds(base_row + off, n)],
                                yq.at[pl.ds(base_row + off, n)])

            @pl.when(sid < NS - 1)
            def _():
                off, n = _SPAN_EXTRA
                pltpu.sync_copy(acc.at[pl.ds(base_row + off, n)],
                                yq.at[pl.ds(base_row + off, n)])


_qtab = jax.ShapeDtypeStruct((NUM_NODES, DQ), jnp.float32)

_layer = pl.kernel(
    _layer_body,
    out_type=(_qtab, _qtab, _qtab, _qtab),
    mesh=_mesh,
    compiler_params=_params,
    scratch_types=[
        *sum([[
            pltpu.VMEM((NSUB, SUB), jnp.int32),
            pltpu.VMEM((NSUB, SUB), jnp.int32),
            pltpu.VMEM((CHUNK,), jnp.float32),
            pltpu.VMEM((CHUNK, DQ), jnp.float32),
        ] for _ in range(4)], []),
        pltpu.VMEM_SHARED((NUM_NODES, DQ), jnp.float32),
        pltpu.SemaphoreType.DMA,
        pltpu.SemaphoreType.DMA,
        pltpu.SemaphoreType.DMA,
    ],
)

BPT = B // (NC * NS)


def _readout_body(*args):
    tabs = args[:16]
    users, items, out = args[16:19]
    uidx, iidx, usum, isum, tmp, sbuf = args[19:]

    cid = lax.axis_index("c")
    sid = lax.axis_index("s")
    wid = sid * NC + cid
    base = wid * BPT

    pltpu.sync_copy(users.at[pl.ds(base, BPT)], uidx)
    pltpu.sync_copy(items.at[pl.ds(base, BPT)], iidx)

    off = jnp.full((L,), NUM_USERS, jnp.int32)
    for k in range(BPT // L):
        iidx[pl.ds(k * L, L)] = iidx[pl.ds(k * L, L)] + off

    for dst_ref, idx_ref in ((usum, uidx), (isum, iidx)):
        for li in range(4):
            for q in range(4):
                t = tabs[li * 4 + q]
                c0 = q * DQ
                pltpu.sync_copy(t.at[idx_ref], tmp)
                if li == 0:
                    def _cp(j, c, c0=c0):
                        dst_ref[j, c0:c0 + 16] = tmp[j, 0:16]
                        return c
                    lax.fori_loop(0, BPT, _cp, 0, unroll=8)
                else:
                    def _add(j, c, c0=c0):
                        dst_ref[j, c0:c0 + 16] = \
                            dst_ref[j, c0:c0 + 16] + tmp[j, 0:16]
                        return c
                    lax.fori_loop(0, BPT, _add, 0, unroll=8)

    iota = lax.iota(jnp.int32, L)
    scale = jnp.full((L,), 1.0 / ((LAYERS + 1) * (LAYERS + 1)), jnp.float32)
    for j0 in range(0, BPT, L):
        ridx = iota + j0

        def _dot(c, acc):
            cv = lax.broadcast(c, (L,))
            return acc + plsc.load_gather(usum, [ridx, cv]) * \
                plsc.load_gather(isum, [ridx, cv])

        acc = lax.fori_loop(0, EMB, _dot, jnp.zeros((L,), jnp.float32),
                            unroll=8)
        sbuf[pl.ds(j0, L)] = acc * scale

    pltpu.sync_copy(sbuf, out.at[pl.ds(base, BPT)])


_readout = pl.kernel(
    _readout_body,
    out_type=jax.ShapeDtypeStruct((B,), jnp.float32),
    mesh=_mesh,
    compiler_params=_params,
    scratch_types=[
        pltpu.VMEM((BPT,), jnp.int32),
        pltpu.VMEM((BPT,), jnp.int32),
        pltpu.VMEM((BPT, EMB), jnp.float32),
        pltpu.VMEM((BPT, EMB), jnp.float32),
        pltpu.VMEM((BPT, DQ), jnp.float32),
        pltpu.VMEM((BPT,), jnp.float32),
    ],
)


def kernel(user_table, item_table, edge_norm, edges, users, items):
    x0 = jnp.concatenate([user_table, item_table], axis=0)
    xq = tuple(x0[:, q * DQ:(q + 1) * DQ] for q in range(4))

    src = edges[:, 0].astype(jnp.int32)
    dst = edges[:, 1].astype(jnp.int32)
    nrm = edge_norm.astype(jnp.float32)
    pad = E_PAD - E
    src2 = jnp.pad(src, (0, pad)).reshape(E_PAD // SUB, SUB)
    dst2 = jnp.pad(dst, (0, pad)).reshape(E_PAD // SUB, SUB)
    nrm1 = jnp.pad(nrm, (0, pad))

    tabs = list(xq)
    for _ in range(LAYERS):
        xq = _layer(*xq, src2, dst2, nrm1)
        tabs.extend(xq)

    return _readout(*tabs, users.astype(jnp.int32), items.astype(jnp.int32))


# speedup vs baseline: 1.6379x; 1.6379x over previous
import jax
import jax.numpy as jnp
from jax import lax
from jax.experimental import pallas as pl
from jax.experimental.pallas import tpu as pltpu
from jax.experimental.pallas import tpu_sc as plsc

NUM_USERS = 25000
NUM_NODES = 50000
EMB = 64
DQ = EMB // 4
E = 800000
B = 4096
LAYERS = 3

NC = 2
NS = 16
L = 16

CHUNK = 1024
SUB = 128
NSUB = CHUNK // SUB
EPT = 51200
NCHUNKS = EPT // CHUNK
NPAIRS = NCHUNKS // 2
E_PAD = EPT * NS
EROWS = (NSUB * SUB * 4) // (DQ * 4)
ROWS_PT = 3128
ROWS_LAST = NUM_NODES - (NS - 1) * ROWS_PT

ZB = 1024
_SPANS_COMMON = ((0, ZB), (ZB, ZB), (2 * ZB, ZB),
                 (3 * ZB, ROWS_LAST - 3 * ZB))
_SPAN_EXTRA = (ROWS_LAST, ROWS_PT - ROWS_LAST)

_mesh = plsc.VectorSubcoreMesh(
    core_axis_name="c", subcore_axis_name="s", num_cores=NC, num_subcores=NS
)
_params = pltpu.CompilerParams(
    needs_layout_passes=False, use_tc_tiling_on_sc=False
)


def _layer_body(x0, x1, x2, x3, src_h, dst_h, nrm_h, y0, y1, y2, y3,
                src_b0, dst_b0, nrm_b0, rows0,
                src_b1, dst_b1, nrm_b1, rows1,
                acc, esem, gsem, ssem):
    rows = rows0
    cid = lax.axis_index("c")
    sid = lax.axis_index("s")

    zero16 = jnp.zeros((L,), jnp.float32)
    base_row = sid * ROWS_PT
    erow_base = sid * (EPT // SUB)
    eflat_base = sid * EPT

    xs = (x0, x1, x2, x3)
    ys = (y0, y1, y2, y3)

    for q in range(4):
        @pl.when(cid == q // 2)
        def _(xq=xs[q], yq=ys[q]):
            def _zrow(j, c):
                rows[j, 0:16] = zero16
                return c
            lax.fori_loop(0, ZB, _zrow, 0, unroll=8)
            for off, n in _SPANS_COMMON:
                pltpu.sync_copy(rows.at[pl.ds(0, n)],
                                acc.at[pl.ds(base_row + off, n)])

            @pl.when(sid < NS - 1)
            def _():
                off, n = _SPAN_EXTRA
                pltpu.sync_copy(rows.at[pl.ds(0, n)],
                                acc.at[pl.ds(base_row + off, n)])
            plsc.subcore_barrier()

            bufs = ((src_b0, dst_b0, nrm_b0, rows0),
                    (src_b1, dst_b1, nrm_b1, rows1))

            def _fire_edges(g, bi):
                sb, db, nb, _ = bufs[bi]
                rb = erow_base + g * NSUB
                return (
                    pltpu.async_copy(src_h.at[pl.ds(rb, NSUB)], sb, esem),
                    pltpu.async_copy(
                        nrm_h.at[pl.ds(eflat_base + g * CHUNK, CHUNK)],
                        nb, esem),
                    pltpu.async_copy(dst_h.at[pl.ds(rb, NSUB)], db, esem),
                )

            def _fire_gathers(bi):
                sb, _, _, rw = bufs[bi]
                return tuple(
                    pltpu.async_copy(xq.at[sb.at[r]],
                                     rw.at[pl.ds(r * SUB, SUB)], gsem)
                    for r in range(NSUB))

            def _scale(bi):
                _, _, nb, rw = bufs[bi]

                def _blk(b, cc):
                    j0 = b * L
                    nvec = nb[pl.ds(j0, L)]
                    for e in range(L):
                        nv = lax.broadcast(nvec[e], (L,))
                        rw[j0 + e, 0:16] = rw[j0 + e, 0:16] * nv
                    return cc
                lax.fori_loop(0, CHUNK // L, _blk, 0, unroll=2)

            def _fire_scatters(bi):
                _, db, _, rw = bufs[bi]
                return tuple(
                    pltpu.async_copy(rw.at[pl.ds(r * SUB, SUB)],
                                     acc.at[db.at[r]], ssem, add=True)
                    for r in range(NSUB))

            def _drain_edges():
                pltpu.make_async_copy(xq.at[pl.ds(0, 3 * EROWS)],
                                      rows0.at[pl.ds(0, 3 * EROWS)],
                                      esem).wait()

            def _drain_chunk(sem, rw):
                pltpu.make_async_copy(xq.at[pl.ds(0, CHUNK)], rw, sem).wait()

            def _pair(t, c):
                _fire_edges(2 * t, 0)
                _fire_edges(2 * t + 1, 1)
                _drain_edges()
                _fire_gathers(0)
                _drain_edges()
                _fire_gathers(1)
                _drain_chunk(gsem, rows0)
                _scale(0)
                _fire_scatters(0)
                _drain_chunk(gsem, rows1)
                _scale(1)
                _fire_scatters(1)
                _drain_chunk(ssem, rows0)
                _drain_chunk(ssem, rows1)
                return c

            lax.fori_loop(0, NPAIRS, _pair, 0)
            plsc.subcore_barrier()

            for off, n in _SPANS_COMMON:
                pltpu.sync_copy(acc.at[pl.ds(base_row + off, n)],
                                yq.at[pl.ds(base_row + off, n)])

            @pl.when(sid < NS - 1)
            def _():
                off, n = _SPAN_EXTRA
                pltpu.sync_copy(acc.at[pl.ds(base_row + off, n)],
                                yq.at[pl.ds(base_row + off, n)])


_qtab = jax.ShapeDtypeStruct((NUM_NODES, DQ), jnp.float32)

_layer = pl.kernel(
    _layer_body,
    out_type=(_qtab, _qtab, _qtab, _qtab),
    mesh=_mesh,
    compiler_params=_params,
    scratch_types=[
        pltpu.VMEM((NSUB, SUB), jnp.int32),
        pltpu.VMEM((NSUB, SUB), jnp.int32),
        pltpu.VMEM((CHUNK,), jnp.float32),
        pltpu.VMEM((CHUNK, DQ), jnp.float32),
        pltpu.VMEM((NSUB, SUB), jnp.int32),
        pltpu.VMEM((NSUB, SUB), jnp.int32),
        pltpu.VMEM((CHUNK,), jnp.float32),
        pltpu.VMEM((CHUNK, DQ), jnp.float32),
        pltpu.VMEM_SHARED((NUM_NODES, DQ), jnp.float32),
        pltpu.SemaphoreType.DMA,
        pltpu.SemaphoreType.DMA,
        pltpu.SemaphoreType.DMA,
    ],
)

BPT = B // (NC * NS)


def _readout_body(*args):
    tabs = args[:16]
    users, items, out = args[16:19]
    uidx, iidx, usum, isum, tmp, sbuf = args[19:]

    cid = lax.axis_index("c")
    sid = lax.axis_index("s")
    wid = sid * NC + cid
    base = wid * BPT

    pltpu.sync_copy(users.at[pl.ds(base, BPT)], uidx)
    pltpu.sync_copy(items.at[pl.ds(base, BPT)], iidx)

    off = jnp.full((L,), NUM_USERS, jnp.int32)
    for k in range(BPT // L):
        iidx[pl.ds(k * L, L)] = iidx[pl.ds(k * L, L)] + off

    for dst_ref, idx_ref in ((usum, uidx), (isum, iidx)):
        for li in range(4):
            for q in range(4):
                t = tabs[li * 4 + q]
                c0 = q * DQ
                pltpu.sync_copy(t.at[idx_ref], tmp)
                if li == 0:
                    def _cp(j, c, c0=c0):
                        dst_ref[j, c0:c0 + 16] = tmp[j, 0:16]
                        return c
                    lax.fori_loop(0, BPT, _cp, 0, unroll=8)
                else:
                    def _add(j, c, c0=c0):
                        dst_ref[j, c0:c0 + 16] = \
                            dst_ref[j, c0:c0 + 16] + tmp[j, 0:16]
                        return c
                    lax.fori_loop(0, BPT, _add, 0, unroll=8)

    iota = lax.iota(jnp.int32, L)
    scale = jnp.full((L,), 1.0 / ((LAYERS + 1) * (LAYERS + 1)), jnp.float32)
    for j0 in range(0, BPT, L):
        ridx = iota + j0

        def _dot(c, acc):
            cv = lax.broadcast(c, (L,))
            return acc + plsc.load_gather(usum, [ridx, cv]) * \
                plsc.load_gather(isum, [ridx, cv])

        acc = lax.fori_loop(0, EMB, _dot, jnp.zeros((L,), jnp.float32),
                            unroll=8)
        sbuf[pl.ds(j0, L)] = acc * scale

    pltpu.sync_copy(sbuf, out.at[pl.ds(base, BPT)])


_readout = pl.kernel(
    _readout_body,
    out_type=jax.ShapeDtypeStruct((B,), jnp.float32),
    mesh=_mesh,
    compiler_params=_params,
    scratch_types=[
        pltpu.VMEM((BPT,), jnp.int32),
        pltpu.VMEM((BPT,), jnp.int32),
        pltpu.VMEM((BPT, EMB), jnp.float32),
        pltpu.VMEM((BPT, EMB), jnp.float32),
        pltpu.VMEM((BPT, DQ), jnp.float32),
        pltpu.VMEM((BPT,), jnp.float32),
    ],
)


def kernel(user_table, item_table, edge_norm, edges, users, items):
    x0 = jnp.concatenate([user_table, item_table], axis=0)
    xq = tuple(x0[:, q * DQ:(q + 1) * DQ] for q in range(4))

    src = edges[:, 0].astype(jnp.int32)
    dst = edges[:, 1].astype(jnp.int32)
    nrm = edge_norm.astype(jnp.float32)
    pad = E_PAD - E
    src2 = jnp.pad(src, (0, pad)).reshape(E_PAD // SUB, SUB)
    dst2 = jnp.pad(dst, (0, pad)).reshape(E_PAD // SUB, SUB)
    nrm1 = jnp.pad(nrm, (0, pad))

    tabs = list(xq)
    for _ in range(LAYERS):
        xq = _layer(*xq, src2, dst2, nrm1)
        tabs.extend(xq)

    return _readout(*tabs, users.astype(jnp.int32), items.astype(jnp.int32))
